# baseline (speedup 1.0000x reference)
import functools

import jax
import jax.numpy as jnp
from jax import lax
from jax.experimental import pallas as pl
from jax.experimental.pallas import tpu as pltpu
from jax.experimental.pallas import tpu_sc as plsc

_M = 2048
_N = 8192
_NC, _NS, _L = 2, 16, 16
_ROWS_PER_W = _M // _NS
_BUF_ROWS = 8
_CHUNKS = _ROWS_PER_W // _BUF_ROWS


def _tc_body(x_ref, w_ref, b_ref, lt_ref, vals_ref):
    pre = jax.lax.dot_general(
        x_ref[...], w_ref[...],
        (((1,), (1,)), ((), ())),
        preferred_element_type=jnp.float32,
    )
    thr = jnp.exp(lt_ref[...])
    vals_ref[...] = ((pre + b_ref[...]) > thr).astype(jnp.float32)


@functools.partial(
    pl.kernel,
    out_type=jax.ShapeDtypeStruct((_M, _N), jnp.int32),
    mesh=plsc.VectorSubcoreMesh(core_axis_name="c", subcore_axis_name="s"),
    scratch_types=[
        pltpu.VMEM((_BUF_ROWS, _N), jnp.int32),
        pltpu.SemaphoreType.DMA,
    ],
)
def _sc_indices(out_hbm, buf, sem):
    sid = lax.axis_index("s")
    cid = lax.axis_index("c")

    def fill(ii, carry):
        i = ii & (_N // _L - 1)
        vec = lax.iota(jnp.int32, _L) + i * _L
        for r in range(_BUF_ROWS):
            buf[r, pl.ds(i * _L, _L)] = vec
        return carry

    lax.fori_loop(0, 4 * (_N // _L), fill, 0)

    @pl.when(cid == 0)
    def _():
        base = sid * _ROWS_PER_W
        handles = [
            pltpu.async_copy(
                buf, out_hbm.at[pl.ds(base + c * _BUF_ROWS, _BUF_ROWS)], sem)
            for c in range(_CHUNKS)
        ]
        for h in handles:
            h.wait()


def kernel(x, W_enc, b_enc, log_threshold, k):
    M, D = x.shape
    N = W_enc.shape[0]
    BM, BN = 2048, 512

    b2 = b_enc.reshape(1, N)
    lt2 = log_threshold.reshape(1, N)

    values = pl.pallas_call(
        _tc_body,
        grid=(M // BM, N // BN),
        in_specs=[
            pl.BlockSpec((BM, D), lambda i, j: (i, 0)),
            pl.BlockSpec((BN, D), lambda i, j: (j, 0)),
            pl.BlockSpec((1, BN), lambda i, j: (0, j)),
            pl.BlockSpec((1, BN), lambda i, j: (0, j)),
        ],
        out_specs=pl.BlockSpec((BM, BN), lambda i, j: (i, j)),
        out_shape=jax.ShapeDtypeStruct((M, N), jnp.float32),
        compiler_params=pltpu.CompilerParams(
            dimension_semantics=("parallel", "parallel")),
    )(x, W_enc, b2, lt2)

    indices = _sc_indices()
    return (values, indices)


# speedup vs baseline: 1.1346x; 1.1346x over previous
import functools

import jax
import jax.numpy as jnp
from jax import lax
from jax.experimental import pallas as pl
from jax.experimental.pallas import tpu as pltpu
from jax.experimental.pallas import tpu_sc as plsc

_M = 2048
_N = 8192
_NC, _NS, _L = 2, 16, 16
_ROWS_PER_W = _M // _NS
_BUF_ROWS = 8
_CHUNKS = _ROWS_PER_W // _BUF_ROWS


def _tc_body(x_ref, w_ref, b_ref, lt_ref, vals_ref, idx_ref):
    pre = jax.lax.dot_general(
        x_ref[...], w_ref[...],
        (((1,), (1,)), ((), ())),
        preferred_element_type=jnp.float32,
    )
    thr = jnp.exp(lt_ref[...])
    vals_ref[...] = ((pre + b_ref[...]) > thr).astype(jnp.float32)
    j = pl.program_id(1)
    bn = idx_ref.shape[1]
    idx_ref[...] = jax.lax.broadcasted_iota(
        jnp.int32, idx_ref.shape, 1) + j * bn


@functools.partial(
    pl.kernel,
    out_type=jax.ShapeDtypeStruct((_M, _N), jnp.int32),
    mesh=plsc.VectorSubcoreMesh(core_axis_name="c", subcore_axis_name="s"),
    scratch_types=[
        pltpu.VMEM((_BUF_ROWS, _N), jnp.int32),
        pltpu.SemaphoreType.DMA,
    ],
)
def _sc_indices(out_hbm, buf, sem):
    sid = lax.axis_index("s")
    cid = lax.axis_index("c")

    def fill(i, carry):
        vec = lax.iota(jnp.int32, _L) + i * _L
        for r in range(_BUF_ROWS):
            buf[r, pl.ds(i * _L, _L)] = vec
        return carry

    lax.fori_loop(0, _N // _L, fill, 0)

    @pl.when(cid == 0)
    def _():
        base = sid * _ROWS_PER_W
        handles = [
            pltpu.async_copy(
                buf, out_hbm.at[pl.ds(base + c * _BUF_ROWS, _BUF_ROWS)], sem)
            for c in range(_CHUNKS)
        ]
        for h in handles:
            h.wait()


def kernel(x, W_enc, b_enc, log_threshold, k):
    M, D = x.shape
    N = W_enc.shape[0]
    BM, BN = 2048, 512

    b2 = b_enc.reshape(1, N)
    lt2 = log_threshold.reshape(1, N)

    values = pl.pallas_call(
        _tc_body,
        grid=(M // BM, N // BN),
        in_specs=[
            pl.BlockSpec((BM, D), lambda i, j: (i, 0)),
            pl.BlockSpec((BN, D), lambda i, j: (j, 0)),
            pl.BlockSpec((1, BN), lambda i, j: (0, j)),
            pl.BlockSpec((1, BN), lambda i, j: (0, j)),
        ],
        out_specs=[
            pl.BlockSpec((BM, BN), lambda i, j: (i, j)),
            pl.BlockSpec((BM, BN), lambda i, j: (i, j)),
        ],
        out_shape=[
            jax.ShapeDtypeStruct((M, N), jnp.float32),
            jax.ShapeDtypeStruct((M, N), jnp.int32),
        ],
        compiler_params=pltpu.CompilerParams(
            dimension_semantics=("parallel", "parallel")),
    )(x, W_enc, b2, lt2)
    values, indices = values
    return (values, indices)


# speedup vs baseline: 1.1413x; 1.0059x over previous
import functools

import jax
import jax.numpy as jnp
from jax import lax
from jax.experimental import pallas as pl
from jax.experimental.pallas import tpu as pltpu
from jax.experimental.pallas import tpu_sc as plsc

_M = 2048
_N = 8192
_NC, _NS, _L = 2, 16, 16
_ROWS_PER_W = _M // _NS
_BUF_ROWS = 8
_CHUNKS = _ROWS_PER_W // _BUF_ROWS


def _tc_body(x1_ref, x2_ref, w_ref, b_ref, lt_ref, vals_ref, idx_ref):
    dk = x1_ref.shape[1]
    dn = (((1,), (1,)), ((), ()))
    pre = jax.lax.dot_general(
        x1_ref[...], w_ref[:, :dk], dn,
        preferred_element_type=jnp.float32,
    ) + jax.lax.dot_general(
        x2_ref[...], w_ref[:, dk:], dn,
        preferred_element_type=jnp.float32,
    )
    thr = jnp.exp(lt_ref[...])
    vals_ref[...] = ((pre + b_ref[...]) > thr).astype(jnp.float32)
    j = pl.program_id(1)
    bn = idx_ref.shape[1]
    idx_ref[...] = jax.lax.broadcasted_iota(
        jnp.int32, idx_ref.shape, 1) + j * bn


@functools.partial(
    pl.kernel,
    out_type=jax.ShapeDtypeStruct((_M, _N), jnp.int32),
    mesh=plsc.VectorSubcoreMesh(core_axis_name="c", subcore_axis_name="s"),
    scratch_types=[
        pltpu.VMEM((_BUF_ROWS, _N), jnp.int32),
        pltpu.SemaphoreType.DMA,
    ],
)
def _sc_indices(out_hbm, buf, sem):
    sid = lax.axis_index("s")
    cid = lax.axis_index("c")

    def fill(i, carry):
        vec = lax.iota(jnp.int32, _L) + i * _L
        for r in range(_BUF_ROWS):
            buf[r, pl.ds(i * _L, _L)] = vec
        return carry

    lax.fori_loop(0, _N // _L, fill, 0)

    @pl.when(cid == 0)
    def _():
        base = sid * _ROWS_PER_W
        handles = [
            pltpu.async_copy(
                buf, out_hbm.at[pl.ds(base + c * _BUF_ROWS, _BUF_ROWS)], sem)
            for c in range(_CHUNKS)
        ]
        for h in handles:
            h.wait()


def kernel(x, W_enc, b_enc, log_threshold, k):
    M, D = x.shape
    N = W_enc.shape[0]
    BM, BN = 2048, 512

    b2 = b_enc.reshape(1, N)
    lt2 = log_threshold.reshape(1, N)

    values = pl.pallas_call(
        _tc_body,
        grid=(M // BM, N // BN),
        in_specs=[
            pl.BlockSpec((BM, D // 2), lambda i, j: (i, 0)),
            pl.BlockSpec((BM, D // 2), lambda i, j: (i, 1)),
            pl.BlockSpec((BN, D), lambda i, j: (j, 0)),
            pl.BlockSpec((1, BN), lambda i, j: (0, j)),
            pl.BlockSpec((1, BN), lambda i, j: (0, j)),
        ],
        out_specs=[
            pl.BlockSpec((BM, BN), lambda i, j: (i, j)),
            pl.BlockSpec((BM, BN), lambda i, j: (i, j)),
        ],
        out_shape=[
            jax.ShapeDtypeStruct((M, N), jnp.float32),
            jax.ShapeDtypeStruct((M, N), jnp.int32),
        ],
        compiler_params=pltpu.CompilerParams(
            dimension_semantics=("parallel", "parallel")),
    )(x, x, W_enc, b2, lt2)
    values, indices = values
    return (values, indices)


# speedup vs baseline: 1.1414x; 1.0001x over previous
import functools

import jax
import jax.numpy as jnp
from jax import lax
from jax.experimental import pallas as pl
from jax.experimental.pallas import tpu as pltpu
from jax.experimental.pallas import tpu_sc as plsc

_M = 2048
_N = 8192
_NC, _NS, _L = 2, 16, 16
_ROWS_PER_W = _M // _NS
_BUF_ROWS = 8
_CHUNKS = _ROWS_PER_W // _BUF_ROWS


def _tc_body(x1_ref, x2_ref, w1_ref, w2_ref, b_ref, lt_ref, vals_ref, idx_ref):
    dn = (((1,), (1,)), ((), ()))
    pre = jax.lax.dot_general(
        x1_ref[...], w1_ref[...], dn,
        preferred_element_type=jnp.float32,
    ) + jax.lax.dot_general(
        x2_ref[...], w2_ref[...], dn,
        preferred_element_type=jnp.float32,
    )
    thr = jnp.exp(lt_ref[...])
    vals_ref[...] = ((pre + b_ref[...]) > thr).astype(jnp.float32)
    j = pl.program_id(1)
    bn = idx_ref.shape[1]
    idx_ref[...] = jax.lax.broadcasted_iota(
        jnp.int32, idx_ref.shape, 1) + j * bn


@functools.partial(
    pl.kernel,
    out_type=jax.ShapeDtypeStruct((_M, _N), jnp.int32),
    mesh=plsc.VectorSubcoreMesh(core_axis_name="c", subcore_axis_name="s"),
    scratch_types=[
        pltpu.VMEM((_BUF_ROWS, _N), jnp.int32),
        pltpu.SemaphoreType.DMA,
    ],
)
def _sc_indices(out_hbm, buf, sem):
    sid = lax.axis_index("s")
    cid = lax.axis_index("c")

    def fill(i, carry):
        vec = lax.iota(jnp.int32, _L) + i * _L
        for r in range(_BUF_ROWS):
            buf[r, pl.ds(i * _L, _L)] = vec
        return carry

    lax.fori_loop(0, _N // _L, fill, 0)

    @pl.when(cid == 0)
    def _():
        base = sid * _ROWS_PER_W
        handles = [
            pltpu.async_copy(
                buf, out_hbm.at[pl.ds(base + c * _BUF_ROWS, _BUF_ROWS)], sem)
            for c in range(_CHUNKS)
        ]
        for h in handles:
            h.wait()


def kernel(x, W_enc, b_enc, log_threshold, k):
    M, D = x.shape
    N = W_enc.shape[0]
    BM, BN = 2048, 512

    b2 = b_enc.reshape(1, N)
    lt2 = log_threshold.reshape(1, N)

    values = pl.pallas_call(
        _tc_body,
        grid=(M // BM, N // BN),
        in_specs=[
            pl.BlockSpec((BM, D // 2), lambda i, j: (i, 0)),
            pl.BlockSpec((BM, D // 2), lambda i, j: (i, 1)),
            pl.BlockSpec((BN, D // 2), lambda i, j: (j, 0)),
            pl.BlockSpec((BN, D // 2), lambda i, j: (j, 1)),
            pl.BlockSpec((1, BN), lambda i, j: (0, j)),
            pl.BlockSpec((1, BN), lambda i, j: (0, j)),
        ],
        out_specs=[
            pl.BlockSpec((BM, BN), lambda i, j: (i, j)),
            pl.BlockSpec((BM, BN), lambda i, j: (i, j)),
        ],
        out_shape=[
            jax.ShapeDtypeStruct((M, N), jnp.float32),
            jax.ShapeDtypeStruct((M, N), jnp.int32),
        ],
        compiler_params=pltpu.CompilerParams(
            dimension_semantics=("parallel", "parallel")),
    )(x, x, W_enc, W_enc, b2, lt2)
    values, indices = values
    return (values, indices)
